# baseline (speedup 1.0000x reference)
import functools

import jax
import jax.numpy as jnp
from jax.experimental import pallas as pl
from jax.experimental.pallas import tpu as pltpu

K_WIN = 4768
SUB = 4
K_BLK = K_WIN // SUB


def _bf16_rne(v):
    u = jax.lax.bitcast_convert_type(v, jnp.uint32)
    lsb = (u >> 16) & jnp.uint32(1)
    u = (u + jnp.uint32(0x7FFF) + lsb) & jnp.uint32(0xFFFF0000)
    return jax.lax.bitcast_convert_type(u, jnp.float32)


def _nn_body(xsq_ref, csq_ref, x_ref, c_ref, out_ref, min_ref, arg_ref,
             *, n_steps, k_real):
    k = pl.program_id(0)
    rem = k_real - k * K_BLK

    row = jax.lax.broadcasted_iota(jnp.int32, (K_BLK, 1), 0)
    c = jnp.where(row < rem, c_ref[...], 0.0)

    mm = jax.lax.dot_general(
        x_ref[...], c, (((1,), (1,)), ((), ())),
        preferred_element_type=jnp.float32)

    csq = csq_ref[...].reshape(1, K_BLK)
    d = (xsq_ref[...] + csq) - 2.0 * mm

    local_min = jnp.min(d, axis=1, keepdims=True)
    col = jax.lax.broadcasted_iota(jnp.int32, d.shape, 1)
    cand = jnp.where(d == local_min, col, jnp.int32(2**31 - 1))
    local_arg = jnp.min(cand, axis=1, keepdims=True) + k * K_BLK

    @pl.when(k == 0)
    def _():
        min_ref[...] = local_min
        arg_ref[...] = local_arg

    @pl.when(k > 0)
    def _():
        better = local_min < min_ref[...]
        min_ref[...] = jnp.where(better, local_min, min_ref[...])
        arg_ref[...] = jnp.where(better, local_arg, arg_ref[...])

    @pl.when((k % SUB) == SUB - 1)
    def _():
        min_ref[...] = _bf16_rne(min_ref[...])

    @pl.when(k == n_steps - 1)
    def _():
        out_ref[...] = arg_ref[...]


def kernel(x, codes):
    b, n, dim = x.shape
    k_real = codes.shape[0]
    q = b * n
    xf = x.reshape(q, dim)

    x_sq = jnp.sum(xf * xf, axis=-1, keepdims=True)
    c_sq = jnp.sum(codes * codes, axis=-1)

    n_steps = (k_real + K_BLK - 1) // K_BLK
    k_pad = n_steps * K_BLK
    c_sq_p = jnp.pad(c_sq, (0, k_pad - k_real),
                     constant_values=jnp.inf).reshape(n_steps, 1, K_BLK)

    out = pl.pallas_call(
        functools.partial(_nn_body, n_steps=n_steps, k_real=k_real),
        grid=(n_steps,),
        in_specs=[
            pl.BlockSpec((q, 1), lambda k: (0, 0)),
            pl.BlockSpec((1, 1, K_BLK), lambda k: (k, 0, 0)),
            pl.BlockSpec((q, dim), lambda k: (0, 0)),
            pl.BlockSpec((K_BLK, dim), lambda k: (k, 0)),
        ],
        out_specs=pl.BlockSpec((q, 1), lambda k: (0, 0)),
        out_shape=jax.ShapeDtypeStruct((q, 1), jnp.int32),
        scratch_shapes=[
            pltpu.VMEM((q, 1), jnp.float32),
            pltpu.VMEM((q, 1), jnp.int32),
        ],
    )(x_sq, c_sq_p, xf, codes)

    return out.reshape(b, n)


# speedup vs baseline: 1.0234x; 1.0234x over previous
import functools

import jax
import jax.numpy as jnp
from jax.experimental import pallas as pl
from jax.experimental.pallas import tpu as pltpu

K_WIN = 4768
SUB = 4
K_BLK = K_WIN // SUB


def _bf16_rne(v):
    u = jax.lax.bitcast_convert_type(v, jnp.uint32)
    lsb = (u >> 16) & jnp.uint32(1)
    u = (u + jnp.uint32(0x7FFF) + lsb) & jnp.uint32(0xFFFF0000)
    return jax.lax.bitcast_convert_type(u, jnp.float32)


def _nn_body(xsq_ref, csq_ref, x_ref, c_ref, out_ref, min_ref, arg_ref,
             *, n_steps, k_real):
    k = pl.program_id(0)
    rem = k_real - k * K_BLK

    row = jax.lax.broadcasted_iota(jnp.int32, (K_BLK, 1), 0)
    c = jnp.where(row < rem, c_ref[...], 0.0)

    mm = jax.lax.dot_general(
        c, x_ref[...], (((1,), (1,)), ((), ())),
        preferred_element_type=jnp.float32)

    csq = csq_ref[...].reshape(K_BLK, 1)
    d = (xsq_ref[...] + csq) - 2.0 * mm

    local_min = jnp.min(d, axis=0, keepdims=True)
    rid = jax.lax.broadcasted_iota(jnp.int32, d.shape, 0)
    cand = jnp.where(d == local_min, rid, jnp.int32(2**31 - 1))
    local_arg = jnp.min(cand, axis=0, keepdims=True) + k * K_BLK

    @pl.when(k == 0)
    def _():
        min_ref[...] = local_min
        arg_ref[...] = local_arg

    @pl.when(k > 0)
    def _():
        better = local_min < min_ref[...]
        min_ref[...] = jnp.where(better, local_min, min_ref[...])
        arg_ref[...] = jnp.where(better, local_arg, arg_ref[...])

    @pl.when((k % SUB) == SUB - 1)
    def _():
        min_ref[...] = _bf16_rne(min_ref[...])

    @pl.when(k == n_steps - 1)
    def _():
        out_ref[...] = arg_ref[...]


def kernel(x, codes):
    b, n, dim = x.shape
    k_real = codes.shape[0]
    q = b * n
    xf = x.reshape(q, dim)

    x_sq = jnp.sum(xf * xf, axis=-1).reshape(1, q)
    c_sq = jnp.sum(codes * codes, axis=-1)

    n_steps = (k_real + K_BLK - 1) // K_BLK
    k_pad = n_steps * K_BLK
    c_sq_p = jnp.pad(c_sq, (0, k_pad - k_real),
                     constant_values=jnp.inf).reshape(n_steps, K_BLK, 1)

    out = pl.pallas_call(
        functools.partial(_nn_body, n_steps=n_steps, k_real=k_real),
        grid=(n_steps,),
        in_specs=[
            pl.BlockSpec((1, q), lambda k: (0, 0)),
            pl.BlockSpec((1, K_BLK, 1), lambda k: (k, 0, 0)),
            pl.BlockSpec((q, dim), lambda k: (0, 0)),
            pl.BlockSpec((K_BLK, dim), lambda k: (k, 0)),
        ],
        out_specs=pl.BlockSpec((1, q), lambda k: (0, 0)),
        out_shape=jax.ShapeDtypeStruct((1, q), jnp.int32),
        scratch_shapes=[
            pltpu.VMEM((1, q), jnp.float32),
            pltpu.VMEM((1, q), jnp.int32),
        ],
    )(x_sq, c_sq_p, xf, codes)

    return out.reshape(b, n)


# speedup vs baseline: 1.3971x; 1.3652x over previous
import functools

import jax
import jax.numpy as jnp
from jax.experimental import pallas as pl
from jax.experimental.pallas import tpu as pltpu

K_WIN = 4768
SUB = 4
K_BLK = K_WIN // SUB


def _bf16_rne(v):
    u = jax.lax.bitcast_convert_type(v, jnp.uint32)
    lsb = (u >> 16) & jnp.uint32(1)
    u = (u + jnp.uint32(0x7FFF) + lsb) & jnp.uint32(0xFFFF0000)
    return jax.lax.bitcast_convert_type(u, jnp.float32)


def _nn_body(xsq_ref, csq_ref, x_ref, c_ref, out_ref, min_ref, arg_ref,
             *, n_steps, k_real):
    k = pl.program_id(0)
    rem = k_real - k * K_BLK

    row = jax.lax.broadcasted_iota(jnp.int32, (K_BLK, 1), 0)
    c = jnp.where(row < rem, c_ref[...], 0.0)

    mm = jax.lax.dot_general(
        c, x_ref[...], (((1,), (1,)), ((), ())),
        preferred_element_type=jnp.float32)

    csq = csq_ref[...].reshape(K_BLK, 1)
    xsq = xsq_ref[...]
    q = xsq.shape[1]

    av = jnp.full((8, q), jnp.inf, jnp.float32)
    ai = jnp.zeros((8, q), jnp.int32)
    for r in range(K_BLK // 8):
        mmc = jax.lax.slice(mm, (8 * r, 0), (8 * r + 8, q))
        csqc = jax.lax.slice(csq, (8 * r, 0), (8 * r + 8, 1))
        dch = (xsq + csqc) + mmc
        keep = dch < av
        av = jnp.minimum(dch, av)
        ai = jnp.where(keep, r, ai)

    gi = ai * 8 + jax.lax.broadcasted_iota(jnp.int32, (8, q), 0)
    local_min = jnp.min(av, axis=0, keepdims=True)
    cand = jnp.where(av == local_min, gi, jnp.int32(2**31 - 1))
    local_arg = jnp.min(cand, axis=0, keepdims=True) + k * K_BLK

    @pl.when(k == 0)
    def _():
        min_ref[...] = local_min
        arg_ref[...] = local_arg

    @pl.when(k > 0)
    def _():
        better = local_min < min_ref[...]
        min_ref[...] = jnp.where(better, local_min, min_ref[...])
        arg_ref[...] = jnp.where(better, local_arg, arg_ref[...])

    @pl.when((k % SUB) == SUB - 1)
    def _():
        min_ref[...] = _bf16_rne(min_ref[...])

    @pl.when(k == n_steps - 1)
    def _():
        out_ref[...] = arg_ref[...]


def kernel(x, codes):
    b, n, dim = x.shape
    k_real = codes.shape[0]
    q = b * n
    xf = x.reshape(q, dim)

    x_sq = jnp.sum(xf * xf, axis=-1).reshape(1, q)
    c_sq = jnp.sum(codes * codes, axis=-1)
    xm2 = -2.0 * xf

    n_steps = (k_real + K_BLK - 1) // K_BLK
    k_pad = n_steps * K_BLK
    c_sq_p = jnp.pad(c_sq, (0, k_pad - k_real),
                     constant_values=jnp.inf).reshape(n_steps, K_BLK, 1)

    out = pl.pallas_call(
        functools.partial(_nn_body, n_steps=n_steps, k_real=k_real),
        grid=(n_steps,),
        in_specs=[
            pl.BlockSpec((1, q), lambda k: (0, 0)),
            pl.BlockSpec((1, K_BLK, 1), lambda k: (k, 0, 0)),
            pl.BlockSpec((q, dim), lambda k: (0, 0)),
            pl.BlockSpec((K_BLK, dim), lambda k: (k, 0)),
        ],
        out_specs=pl.BlockSpec((1, q), lambda k: (0, 0)),
        out_shape=jax.ShapeDtypeStruct((1, q), jnp.int32),
        scratch_shapes=[
            pltpu.VMEM((1, q), jnp.float32),
            pltpu.VMEM((1, q), jnp.int32),
        ],
    )(x_sq, c_sq_p, xm2, codes)

    return out.reshape(b, n)


# speedup vs baseline: 1.4963x; 1.0710x over previous
import functools

import jax
import jax.numpy as jnp
from jax.experimental import pallas as pl
from jax.experimental.pallas import tpu as pltpu

K_WIN = 4768
SUB = 2
K_BLK = K_WIN // SUB


def _bf16_rne(v):
    u = jax.lax.bitcast_convert_type(v, jnp.uint32)
    lsb = (u >> 16) & jnp.uint32(1)
    u = (u + jnp.uint32(0x7FFF) + lsb) & jnp.uint32(0xFFFF0000)
    return jax.lax.bitcast_convert_type(u, jnp.float32)


def _nn_body(xsq_ref, csq_ref, x_ref, c_ref, out_ref, min_ref, arg_ref,
             *, n_steps, k_real):
    k = pl.program_id(0)
    rem = k_real - k * K_BLK

    row = jax.lax.broadcasted_iota(jnp.int32, (K_BLK, 1), 0)
    c = jnp.where(row < rem, c_ref[...], 0.0)

    mm = jax.lax.dot_general(
        c, x_ref[...], (((1,), (1,)), ((), ())),
        preferred_element_type=jnp.float32)

    csq = csq_ref[...].reshape(K_BLK, 1)
    xsq = xsq_ref[...]
    q = xsq.shape[1]

    av = jnp.full((8, q), jnp.inf, jnp.float32)
    ai = jnp.zeros((8, q), jnp.int32)
    for r in range(K_BLK // 8):
        mmc = jax.lax.slice(mm, (8 * r, 0), (8 * r + 8, q))
        csqc = jax.lax.slice(csq, (8 * r, 0), (8 * r + 8, 1))
        dch = (xsq + csqc) + mmc
        keep = dch < av
        av = jnp.minimum(dch, av)
        ai = jnp.where(keep, r, ai)

    gi = ai * 8 + jax.lax.broadcasted_iota(jnp.int32, (8, q), 0)
    local_min = jnp.min(av, axis=0, keepdims=True)
    cand = jnp.where(av == local_min, gi, jnp.int32(2**31 - 1))
    local_arg = jnp.min(cand, axis=0, keepdims=True) + k * K_BLK

    @pl.when(k == 0)
    def _():
        min_ref[...] = local_min
        arg_ref[...] = local_arg

    @pl.when(k > 0)
    def _():
        better = local_min < min_ref[...]
        min_ref[...] = jnp.where(better, local_min, min_ref[...])
        arg_ref[...] = jnp.where(better, local_arg, arg_ref[...])

    @pl.when((k % SUB) == SUB - 1)
    def _():
        min_ref[...] = _bf16_rne(min_ref[...])

    @pl.when(k == n_steps - 1)
    def _():
        out_ref[...] = arg_ref[...]


def kernel(x, codes):
    b, n, dim = x.shape
    k_real = codes.shape[0]
    q = b * n
    xf = x.reshape(q, dim)

    x_sq = jnp.sum(xf * xf, axis=-1).reshape(1, q)
    c_sq = jnp.sum(codes * codes, axis=-1)
    xm2 = -2.0 * xf

    n_steps = (k_real + K_BLK - 1) // K_BLK
    k_pad = n_steps * K_BLK
    c_sq_p = jnp.pad(c_sq, (0, k_pad - k_real),
                     constant_values=jnp.inf).reshape(n_steps, K_BLK, 1)

    out = pl.pallas_call(
        functools.partial(_nn_body, n_steps=n_steps, k_real=k_real),
        grid=(n_steps,),
        in_specs=[
            pl.BlockSpec((1, q), lambda k: (0, 0)),
            pl.BlockSpec((1, K_BLK, 1), lambda k: (k, 0, 0)),
            pl.BlockSpec((q, dim), lambda k: (0, 0)),
            pl.BlockSpec((K_BLK, dim), lambda k: (k, 0)),
        ],
        out_specs=pl.BlockSpec((1, q), lambda k: (0, 0)),
        out_shape=jax.ShapeDtypeStruct((1, q), jnp.int32),
        scratch_shapes=[
            pltpu.VMEM((1, q), jnp.float32),
            pltpu.VMEM((1, q), jnp.int32),
        ],
    )(x_sq, c_sq_p, xm2, codes)

    return out.reshape(b, n)
